# baseline (speedup 1.0000x reference)
import functools

import jax
import jax.numpy as jnp
from jax import lax
from jax.experimental import pallas as pl
from jax.experimental.pallas import tpu as pltpu
from jax.experimental.pallas import tpu_sc as plsc

BATCH = 16384
EMBED_DIM = 32
BN_EPS = 1e-5

_NC = 2
_NS = 16
_NW = _NC * _NS
_BPW = BATCH // _NW
_CHUNK = 128
_NCHUNK = _BPW // _CHUNK

_BLK = 2048


def _sc_gather4(uidx, iidx, ug_t, ig_t, um_t, im_t):
    mesh = plsc.VectorSubcoreMesh(core_axis_name="c", subcore_axis_name="s")
    out_sh = jax.ShapeDtypeStruct((_NW, _BPW, EMBED_DIM), jnp.float32)

    @functools.partial(
        pl.kernel,
        mesh=mesh,
        out_type=[out_sh] * 4,
        compiler_params=pltpu.CompilerParams(use_tc_tiling_on_sc=False),
        scratch_types=[
            pltpu.VMEM((_NCHUNK, _CHUNK), jnp.int32),
            pltpu.VMEM((_NCHUNK, _CHUNK), jnp.int32),
            pltpu.VMEM((_BPW, EMBED_DIM), jnp.float32),
            pltpu.VMEM((_BPW, EMBED_DIM), jnp.float32),
            pltpu.VMEM((_BPW, EMBED_DIM), jnp.float32),
            pltpu.VMEM((_BPW, EMBED_DIM), jnp.float32),
            pltpu.SemaphoreType.DMA,
            pltpu.SemaphoreType.DMA,
            pltpu.SemaphoreType.DMA,
            pltpu.SemaphoreType.DMA,
        ],
    )
    def k(uidx_h, iidx_h, ug_h, ig_h, um_h, im_h,
          o_ug, o_ig, o_um, o_im,
          uv, iv, b0, b1, b2, b3, s0, s1, s2, s3):
        wid = lax.axis_index("s") * _NC + lax.axis_index("c")
        pltpu.sync_copy(uidx_h.at[wid], uv)
        pltpu.sync_copy(iidx_h.at[wid], iv)
        copies = []
        for j in range(_NCHUNK):
            sl = pl.ds(j * _CHUNK, _CHUNK)
            copies.append(pltpu.async_copy(ug_h.at[uv.at[j]], b0.at[sl], s0))
            copies.append(pltpu.async_copy(ig_h.at[iv.at[j]], b1.at[sl], s1))
            copies.append(pltpu.async_copy(um_h.at[uv.at[j]], b2.at[sl], s2))
            copies.append(pltpu.async_copy(im_h.at[iv.at[j]], b3.at[sl], s3))
        for c in copies:
            c.wait()
        pltpu.sync_copy(b0, o_ug.at[wid])
        pltpu.sync_copy(b1, o_ig.at[wid])
        pltpu.sync_copy(b2, o_um.at[wid])
        pltpu.sync_copy(b3, o_im.at[wid])

    return k(uidx, iidx, ug_t, ig_t, um_t, im_t)


def _tc_body(ug_ref, ig_ref, um_ref, im_ref,
             w0_ref, b0_ref, w1_ref, b1_ref, w2_ref, b2_ref, w3_ref, b3_ref,
             wpg_ref, wpm_ref, bp_ref, out_ref):
    ug = ug_ref[...]
    ig = ig_ref[...]
    un = jnp.sqrt(jnp.sum(ug * ug, axis=1, keepdims=True))
    vn = jnp.sqrt(jnp.sum(ig * ig, axis=1, keepdims=True))
    gmf = (ug / jnp.maximum(un, 1e-12)) * (ig / jnp.maximum(vn, 1e-12))
    h = jnp.concatenate([um_ref[...], im_ref[...]], axis=1)
    for w_ref, b_ref in ((w0_ref, b0_ref), (w1_ref, b1_ref),
                         (w2_ref, b2_ref), (w3_ref, b3_ref)):
        h = jnp.dot(h, w_ref[...], preferred_element_type=jnp.float32)
        h = jnp.maximum(h + b_ref[...], 0.0)
    pred = (jnp.dot(gmf, wpg_ref[...], preferred_element_type=jnp.float32)
            + jnp.dot(h, wpm_ref[...], preferred_element_type=jnp.float32)
            + bp_ref[...])
    out_ref[...] = 1.0 / (1.0 + jnp.exp(-pred))


def kernel(user_indices, item_indices, user_emb_gmf, item_emb_gmf,
           user_emb_mlp, item_emb_mlp,
           W0, b0, gamma0, beta0, W1, b1, gamma1, beta1,
           W2, b2, gamma2, beta2, W3, b3, gamma3, beta3,
           Wp, bp):
    uidx = user_indices.astype(jnp.int32).reshape(_NW, _NCHUNK, _CHUNK)
    iidx = item_indices.astype(jnp.int32).reshape(_NW, _NCHUNK, _CHUNK)

    g_ug, g_ig, g_um, g_im = _sc_gather4(
        uidx, iidx, user_emb_gmf, item_emb_gmf, user_emb_mlp, item_emb_mlp)
    g_ug = g_ug.reshape(BATCH, EMBED_DIM)
    g_ig = g_ig.reshape(BATCH, EMBED_DIM)
    g_um = g_um.reshape(BATCH, EMBED_DIM)
    g_im = g_im.reshape(BATCH, EMBED_DIM)

    inv = 1.0 / jnp.sqrt(jnp.float32(1.0 + BN_EPS))
    ws, bs = [], []
    for W, b, g, be in ((W0, b0, gamma0, beta0), (W1, b1, gamma1, beta1),
                        (W2, b2, gamma2, beta2), (W3, b3, gamma3, beta3)):
        s = g * inv
        ws.append(W * s[None, :])
        bs.append((b * s + be)[None, :])
    wpg = Wp[:EMBED_DIM, :]
    wpm = Wp[EMBED_DIM:, :]
    bp2 = bp[None, :]

    grid = BATCH // _BLK
    row_spec = pl.BlockSpec((_BLK, EMBED_DIM), lambda i: (i, 0))
    full = lambda a: pl.BlockSpec(a.shape, lambda i: (0,) * a.ndim)
    wspecs = []
    for w, b in zip(ws, bs):
        wspecs += [full(w), full(b)]

    out = pl.pallas_call(
        _tc_body,
        grid=(grid,),
        in_specs=[row_spec, row_spec, row_spec, row_spec] + wspecs
                 + [full(wpg), full(wpm), full(bp2)],
        out_specs=pl.BlockSpec((_BLK, 1), lambda i: (i, 0)),
        out_shape=jax.ShapeDtypeStruct((BATCH, 1), jnp.float32),
    )(g_ug, g_ig, g_um, g_im,
      ws[0], bs[0], ws[1], bs[1], ws[2], bs[2], ws[3], bs[3],
      wpg, wpm, bp2)
    return out


# speedup vs baseline: 1.0003x; 1.0003x over previous
import functools

import jax
import jax.numpy as jnp
from jax import lax
from jax.experimental import pallas as pl
from jax.experimental.pallas import tpu as pltpu
from jax.experimental.pallas import tpu_sc as plsc

BATCH = 16384
EMBED_DIM = 32
BN_EPS = 1e-5

_NC = 2
_NS = 16
_NW = _NC * _NS
_BPW = BATCH // _NW
_CHUNK = 128
_NCHUNK = _BPW // _CHUNK

_BLK = 2048


def _sc_gather4(uidx, iidx, ug_t, ig_t, um_t, im_t):
    mesh = plsc.VectorSubcoreMesh(core_axis_name="c", subcore_axis_name="s")
    out_sh = jax.ShapeDtypeStruct((BATCH, EMBED_DIM), jnp.float32)

    @functools.partial(
        pl.kernel,
        mesh=mesh,
        out_type=[out_sh] * 4,
        compiler_params=pltpu.CompilerParams(use_tc_tiling_on_sc=False),
        scratch_types=[
            pltpu.VMEM((_BPW,), jnp.int32),
            pltpu.VMEM((_BPW,), jnp.int32),
            pltpu.VMEM((_BPW, EMBED_DIM), jnp.float32),
            pltpu.VMEM((_BPW, EMBED_DIM), jnp.float32),
            pltpu.VMEM((_BPW, EMBED_DIM), jnp.float32),
            pltpu.VMEM((_BPW, EMBED_DIM), jnp.float32),
            pltpu.SemaphoreType.DMA,
            pltpu.SemaphoreType.DMA,
            pltpu.SemaphoreType.DMA,
            pltpu.SemaphoreType.DMA,
        ],
    )
    def k(uidx_h, iidx_h, ug_h, ig_h, um_h, im_h,
          o_ug, o_ig, o_um, o_im,
          uv, iv, b0, b1, b2, b3, s0, s1, s2, s3):
        wid = lax.axis_index("s") * _NC + lax.axis_index("c")
        base = wid * _BPW
        pltpu.sync_copy(uidx_h.at[pl.ds(base, _BPW)], uv)
        pltpu.sync_copy(iidx_h.at[pl.ds(base, _BPW)], iv)
        copies = []
        for j in range(_NCHUNK):
            sl = pl.ds(j * _CHUNK, _CHUNK)
            copies.append(pltpu.async_copy(ug_h.at[uv.at[sl]], b0.at[sl], s0))
            copies.append(pltpu.async_copy(ig_h.at[iv.at[sl]], b1.at[sl], s1))
            copies.append(pltpu.async_copy(um_h.at[uv.at[sl]], b2.at[sl], s2))
            copies.append(pltpu.async_copy(im_h.at[iv.at[sl]], b3.at[sl], s3))
        for c in copies:
            c.wait()
        osl = pl.ds(base, _BPW)
        pltpu.sync_copy(b0, o_ug.at[osl])
        pltpu.sync_copy(b1, o_ig.at[osl])
        pltpu.sync_copy(b2, o_um.at[osl])
        pltpu.sync_copy(b3, o_im.at[osl])

    return k(uidx, iidx, ug_t, ig_t, um_t, im_t)


def _tc_body(ug_ref, ig_ref, um_ref, im_ref,
             w0_ref, b0_ref, w1_ref, b1_ref, w2_ref, b2_ref, w3_ref, b3_ref,
             wpg_ref, wpm_ref, bp_ref, out_ref):
    ug = ug_ref[...]
    ig = ig_ref[...]
    un = jnp.sqrt(jnp.sum(ug * ug, axis=1, keepdims=True))
    vn = jnp.sqrt(jnp.sum(ig * ig, axis=1, keepdims=True))
    gmf = (ug / jnp.maximum(un, 1e-12)) * (ig / jnp.maximum(vn, 1e-12))
    h = jnp.concatenate([um_ref[...], im_ref[...]], axis=1)
    for w_ref, b_ref in ((w0_ref, b0_ref), (w1_ref, b1_ref),
                         (w2_ref, b2_ref), (w3_ref, b3_ref)):
        h = jnp.dot(h, w_ref[...], preferred_element_type=jnp.float32)
        h = jnp.maximum(h + b_ref[...], 0.0)
    pred = (jnp.dot(gmf, wpg_ref[...], preferred_element_type=jnp.float32)
            + jnp.dot(h, wpm_ref[...], preferred_element_type=jnp.float32)
            + bp_ref[...])
    out_ref[...] = 1.0 / (1.0 + jnp.exp(-pred))


def kernel(user_indices, item_indices, user_emb_gmf, item_emb_gmf,
           user_emb_mlp, item_emb_mlp,
           W0, b0, gamma0, beta0, W1, b1, gamma1, beta1,
           W2, b2, gamma2, beta2, W3, b3, gamma3, beta3,
           Wp, bp):
    uidx = user_indices.astype(jnp.int32)
    iidx = item_indices.astype(jnp.int32)

    g_ug, g_ig, g_um, g_im = _sc_gather4(
        uidx, iidx, user_emb_gmf, item_emb_gmf, user_emb_mlp, item_emb_mlp)

    inv = 1.0 / jnp.sqrt(jnp.float32(1.0 + BN_EPS))
    ws, bs = [], []
    for W, b, g, be in ((W0, b0, gamma0, beta0), (W1, b1, gamma1, beta1),
                        (W2, b2, gamma2, beta2), (W3, b3, gamma3, beta3)):
        s = g * inv
        ws.append(W * s[None, :])
        bs.append((b * s + be)[None, :])
    wpg = Wp[:EMBED_DIM, :]
    wpm = Wp[EMBED_DIM:, :]
    bp2 = bp[None, :]

    grid = BATCH // _BLK
    row_spec = pl.BlockSpec((_BLK, EMBED_DIM), lambda i: (i, 0))
    full = lambda a: pl.BlockSpec(a.shape, lambda i: (0,) * a.ndim)
    wspecs = []
    for w, b in zip(ws, bs):
        wspecs += [full(w), full(b)]

    out = pl.pallas_call(
        _tc_body,
        grid=(grid,),
        in_specs=[row_spec, row_spec, row_spec, row_spec] + wspecs
                 + [full(wpg), full(wpm), full(bp2)],
        out_specs=pl.BlockSpec((_BLK, 1), lambda i: (i, 0)),
        out_shape=jax.ShapeDtypeStruct((BATCH, 1), jnp.float32),
    )(g_ug, g_ig, g_um, g_im,
      ws[0], bs[0], ws[1], bs[1], ws[2], bs[2], ws[3], bs[3],
      wpg, wpm, bp2)
    return out


# speedup vs baseline: 3.7616x; 3.7604x over previous
import functools

import jax
import jax.numpy as jnp
from jax import lax
from jax.experimental import pallas as pl
from jax.experimental.pallas import tpu as pltpu
from jax.experimental.pallas import tpu_sc as plsc

BATCH = 16384
EMBED_DIM = 32
BN_EPS = 1e-5

_NC = 2
_NS = 16
_NW = _NC * _NS
_BPW = BATCH // _NW
_CHUNK = 128
_NCHUNK = _BPW // _CHUNK

_BLK = 2048


def _sc_gather4(uidx, iidx, ug_t, ig_t, um_t, im_t):
    mesh = plsc.VectorSubcoreMesh(core_axis_name="c", subcore_axis_name="s")
    out_sh = jax.ShapeDtypeStruct((EMBED_DIM, BATCH), jnp.float32)
    _K = 3
    tile_t = pltpu.VMEM((_K, EMBED_DIM, 128), jnp.float32)
    col_t = pltpu.VMEM((EMBED_DIM, _BPW), jnp.float32)

    @functools.partial(
        pl.kernel,
        mesh=mesh,
        out_type=[out_sh] * 4,
        compiler_params=pltpu.CompilerParams(needs_layout_passes=False),
        scratch_types=[
            pltpu.VMEM((_BPW + 16,), jnp.int32),
            pltpu.VMEM((_BPW + 16,), jnp.int32),
            [tile_t] * 4,
            [col_t] * 4,
            [pltpu.SemaphoreType.DMA((_K,))] * 4,
        ],
    )
    def k(uidx_h, iidx_h, ug_h, ig_h, um_h, im_h,
          o_ug, o_ig, o_um, o_im,
          uv, iv, tiles, cols, sems):
        wid = lax.axis_index("s") * _NC + lax.axis_index("c")
        base = wid * _BPW
        pltpu.sync_copy(uidx_h.at[pl.ds(base, _BPW)], uv.at[pl.ds(0, _BPW)])
        pltpu.sync_copy(iidx_h.at[pl.ds(base, _BPW)], iv.at[pl.ds(0, _BPW)])
        tabs = (ug_h, ig_h, um_h, im_h)
        idxs = (uv, iv, uv, iv)
        outs = (o_ug, o_ig, o_um, o_im)
        rows0 = lax.iota(jnp.int32, 16)
        rows1 = rows0 + 16

        def body(n, _):
            @pl.when(n < _BPW)
            def _fire():
                slot = lax.rem(n, _K)
                ru = uv[pl.ds(n, 16)][0]
                ri = iv[pl.ds(n, 16)][0]
                tcu = pl.multiple_of((ru // 128) * 128, 128)
                tci = pl.multiple_of((ri // 128) * 128, 128)
                for t, tc in ((0, tcu), (1, tci), (2, tcu), (3, tci)):
                    pltpu.async_copy(tabs[t].at[:, pl.ds(tc, 128)],
                                     tiles[t].at[slot], sems[t].at[slot])

            m = n - (_K - 1)

            @pl.when(m >= 0)
            def _extract():
                mslot = lax.rem(m, _K)
                sv = jnp.full((16,), mslot, dtype=jnp.int32)
                mv = jnp.full((16,), m, dtype=jnp.int32)
                lu = jnp.full((16,), uv[pl.ds(m, 16)][0] % 128, jnp.int32)
                li = jnp.full((16,), iv[pl.ds(m, 16)][0] % 128, jnp.int32)
                for t, lane in ((0, lu), (1, li), (2, lu), (3, li)):
                    pltpu.make_async_copy(
                        tabs[t].at[:, pl.ds(0, 128)],
                        tiles[t].at[0], sems[t].at[mslot]).wait()
                    for rows in (rows0, rows1):
                        v = plsc.load_gather(tiles[t], [sv, rows, lane])
                        plsc.store_scatter(cols[t], [rows, mv], v)

            return _

        lax.fori_loop(0, _BPW + _K - 1, body, 0)
        for t in range(4):
            pltpu.sync_copy(cols[t], outs[t].at[:, pl.ds(base, _BPW)])

    return k(uidx, iidx, ug_t, ig_t, um_t, im_t)


def _tc_body(ug_ref, ig_ref, um_ref, im_ref,
             w0_ref, b0_ref, w1_ref, b1_ref, w2_ref, b2_ref, w3_ref, b3_ref,
             wpg_ref, wpm_ref, bp_ref, out_ref):
    ug = ug_ref[...]
    ig = ig_ref[...]
    un = jnp.sqrt(jnp.sum(ug * ug, axis=0, keepdims=True))
    vn = jnp.sqrt(jnp.sum(ig * ig, axis=0, keepdims=True))
    gmf = (ug / jnp.maximum(un, 1e-12)) * (ig / jnp.maximum(vn, 1e-12))
    h = jnp.concatenate([um_ref[...], im_ref[...]], axis=0)
    for w_ref, b_ref in ((w0_ref, b0_ref), (w1_ref, b1_ref),
                         (w2_ref, b2_ref), (w3_ref, b3_ref)):
        h = jnp.dot(w_ref[...], h, preferred_element_type=jnp.float32)
        h = jnp.maximum(h + b_ref[...], 0.0)
    pred = (jnp.dot(wpg_ref[...], gmf, preferred_element_type=jnp.float32)
            + jnp.dot(wpm_ref[...], h, preferred_element_type=jnp.float32)
            + bp_ref[...])
    out_ref[...] = 1.0 / (1.0 + jnp.exp(-pred))


def kernel(user_indices, item_indices, user_emb_gmf, item_emb_gmf,
           user_emb_mlp, item_emb_mlp,
           W0, b0, gamma0, beta0, W1, b1, gamma1, beta1,
           W2, b2, gamma2, beta2, W3, b3, gamma3, beta3,
           Wp, bp):
    uidx = user_indices.astype(jnp.int32)
    iidx = item_indices.astype(jnp.int32)

    tabs = [t.T for t in
            (user_emb_gmf, item_emb_gmf, user_emb_mlp, item_emb_mlp)]
    g_ug, g_ig, g_um, g_im = _sc_gather4(uidx, iidx, *tabs)

    inv = 1.0 / jnp.sqrt(jnp.float32(1.0 + BN_EPS))
    ws, bs = [], []
    for W, b, g, be in ((W0, b0, gamma0, beta0), (W1, b1, gamma1, beta1),
                        (W2, b2, gamma2, beta2), (W3, b3, gamma3, beta3)):
        s = g * inv
        ws.append((W * s[None, :]).T)
        bs.append((b * s + be)[:, None])
    wpg = Wp[:EMBED_DIM, :].T
    wpm = Wp[EMBED_DIM:, :].T
    bp2 = bp[:, None]

    grid = BATCH // _BLK
    col_spec = pl.BlockSpec((EMBED_DIM, _BLK), lambda i: (0, i))
    full = lambda a: pl.BlockSpec(a.shape, lambda i: (0,) * a.ndim)
    wspecs = []
    for w, b in zip(ws, bs):
        wspecs += [full(w), full(b)]

    out = pl.pallas_call(
        _tc_body,
        grid=(grid,),
        in_specs=[col_spec, col_spec, col_spec, col_spec]
                 + wspecs + [full(wpg), full(wpm), full(bp2)],
        out_specs=pl.BlockSpec((1, _BLK), lambda i: (0, i)),
        out_shape=jax.ShapeDtypeStruct((1, BATCH), jnp.float32),
    )(g_ug, g_ig, g_um, g_im,
      ws[0], bs[0], ws[1], bs[1], ws[2], bs[2], ws[3], bs[3],
      wpg, wpm, bp2)
    return out.reshape(BATCH, 1)


# speedup vs baseline: 3.7848x; 1.0062x over previous
import functools

import jax
import jax.numpy as jnp
from jax import lax
from jax.experimental import pallas as pl
from jax.experimental.pallas import tpu as pltpu
from jax.experimental.pallas import tpu_sc as plsc

BATCH = 16384
EMBED_DIM = 32
BN_EPS = 1e-5

_NC = 2
_NS = 16
_NW = _NC * _NS
_BPW = BATCH // _NW
_CHUNK = 128
_NCHUNK = _BPW // _CHUNK

_BLK = 2048


def _sc_gather4(uidx, iidx, ug_t, ig_t, um_t, im_t):
    mesh = plsc.VectorSubcoreMesh(core_axis_name="c", subcore_axis_name="s")
    out_sh = jax.ShapeDtypeStruct((EMBED_DIM, BATCH), jnp.float32)
    _K = 5
    _CB = 128
    _NB = _BPW // _CB
    tile_t = pltpu.VMEM((_K, EMBED_DIM, 128), jnp.float32)
    col_t = pltpu.VMEM((2, EMBED_DIM, _CB), jnp.float32)

    @functools.partial(
        pl.kernel,
        mesh=mesh,
        out_type=[out_sh] * 4,
        compiler_params=pltpu.CompilerParams(needs_layout_passes=False),
        scratch_types=[
            pltpu.VMEM((_BPW + 16,), jnp.int32),
            pltpu.VMEM((_BPW + 16,), jnp.int32),
            [tile_t] * 4,
            [col_t] * 4,
            [pltpu.SemaphoreType.DMA((_K,))] * 4,
            [pltpu.SemaphoreType.DMA] * 4,
        ],
    )
    def k(uidx_h, iidx_h, ug_h, ig_h, um_h, im_h,
          o_ug, o_ig, o_um, o_im,
          uv, iv, tiles, cols, sems, wsems):
        wid = lax.axis_index("s") * _NC + lax.axis_index("c")
        base = wid * _BPW
        pltpu.sync_copy(uidx_h.at[pl.ds(base, _BPW)], uv.at[pl.ds(0, _BPW)])
        pltpu.sync_copy(iidx_h.at[pl.ds(base, _BPW)], iv.at[pl.ds(0, _BPW)])
        tabs = (ug_h, ig_h, um_h, im_h)
        outs = (o_ug, o_ig, o_um, o_im)
        rows0 = lax.iota(jnp.int32, 16)
        rows1 = rows0 + 16

        def body(n, _):
            @pl.when(n < _BPW)
            def _fire():
                slot = lax.rem(n, _K)
                ru = uv[pl.ds(n, 16)][0]
                ri = iv[pl.ds(n, 16)][0]
                tcu = pl.multiple_of((ru // 128) * 128, 128)
                tci = pl.multiple_of((ri // 128) * 128, 128)
                for t, tc in ((0, tcu), (1, tci), (2, tcu), (3, tci)):
                    pltpu.async_copy(tabs[t].at[:, pl.ds(tc, 128)],
                                     tiles[t].at[slot], sems[t].at[slot])

            m = n - (_K - 1)

            @pl.when(m >= 0)
            def _extract():
                mslot = lax.rem(m, _K)
                blk = lax.rem(m // _CB, 2)
                sv = jnp.full((16,), mslot, dtype=jnp.int32)
                bv = jnp.full((16,), blk, dtype=jnp.int32)
                cv = jnp.full((16,), lax.rem(m, _CB), dtype=jnp.int32)
                lu = jnp.full((16,), uv[pl.ds(m, 16)][0] % 128, jnp.int32)
                li = jnp.full((16,), iv[pl.ds(m, 16)][0] % 128, jnp.int32)

                @pl.when(jnp.logical_and(lax.rem(m, _CB) == 0, m >= 2 * _CB))
                def _wait_prev():
                    for t in range(4):
                        pltpu.make_async_copy(
                            tabs[t].at[:, pl.ds(0, _CB)],
                            cols[t].at[0], wsems[t]).wait()

                for t, lane in ((0, lu), (1, li), (2, lu), (3, li)):
                    pltpu.make_async_copy(
                        tabs[t].at[:, pl.ds(0, 128)],
                        tiles[t].at[0], sems[t].at[mslot]).wait()
                    for rows in (rows0, rows1):
                        v = plsc.load_gather(tiles[t], [sv, rows, lane])
                        plsc.store_scatter(cols[t], [bv, rows, cv], v)

                @pl.when(lax.rem(m, _CB) == _CB - 1)
                def _flush():
                    off = pl.multiple_of(base + (m - (_CB - 1)), _CB)
                    for t in range(4):
                        pltpu.async_copy(cols[t].at[blk],
                                         outs[t].at[:, pl.ds(off, _CB)],
                                         wsems[t])

            return _

        lax.fori_loop(0, _BPW + _K - 1, body, 0)
        for t in range(4):
            for _i in range(2):
                pltpu.make_async_copy(
                    tabs[t].at[:, pl.ds(0, _CB)],
                    cols[t].at[0], wsems[t]).wait()

    return k(uidx, iidx, ug_t, ig_t, um_t, im_t)


def _tc_body(ug_ref, ig_ref, um_ref, im_ref,
             w0_ref, b0_ref, w1_ref, b1_ref, w2_ref, b2_ref, w3_ref, b3_ref,
             wpg_ref, wpm_ref, bp_ref, out_ref):
    ug = ug_ref[...]
    ig = ig_ref[...]
    un = jnp.sqrt(jnp.sum(ug * ug, axis=0, keepdims=True))
    vn = jnp.sqrt(jnp.sum(ig * ig, axis=0, keepdims=True))
    gmf = (ug / jnp.maximum(un, 1e-12)) * (ig / jnp.maximum(vn, 1e-12))
    h = jnp.concatenate([um_ref[...], im_ref[...]], axis=0)
    for w_ref, b_ref in ((w0_ref, b0_ref), (w1_ref, b1_ref),
                         (w2_ref, b2_ref), (w3_ref, b3_ref)):
        h = jnp.dot(w_ref[...], h, preferred_element_type=jnp.float32)
        h = jnp.maximum(h + b_ref[...], 0.0)
    pred = (jnp.dot(wpg_ref[...], gmf, preferred_element_type=jnp.float32)
            + jnp.dot(wpm_ref[...], h, preferred_element_type=jnp.float32)
            + bp_ref[...])
    out_ref[...] = 1.0 / (1.0 + jnp.exp(-pred))


def kernel(user_indices, item_indices, user_emb_gmf, item_emb_gmf,
           user_emb_mlp, item_emb_mlp,
           W0, b0, gamma0, beta0, W1, b1, gamma1, beta1,
           W2, b2, gamma2, beta2, W3, b3, gamma3, beta3,
           Wp, bp):
    uidx = user_indices.astype(jnp.int32)
    iidx = item_indices.astype(jnp.int32)

    tabs = [t.T for t in
            (user_emb_gmf, item_emb_gmf, user_emb_mlp, item_emb_mlp)]
    g_ug, g_ig, g_um, g_im = _sc_gather4(uidx, iidx, *tabs)

    inv = 1.0 / jnp.sqrt(jnp.float32(1.0 + BN_EPS))
    ws, bs = [], []
    for W, b, g, be in ((W0, b0, gamma0, beta0), (W1, b1, gamma1, beta1),
                        (W2, b2, gamma2, beta2), (W3, b3, gamma3, beta3)):
        s = g * inv
        ws.append((W * s[None, :]).T)
        bs.append((b * s + be)[:, None])
    wpg = Wp[:EMBED_DIM, :].T
    wpm = Wp[EMBED_DIM:, :].T
    bp2 = bp[:, None]

    grid = BATCH // _BLK
    col_spec = pl.BlockSpec((EMBED_DIM, _BLK), lambda i: (0, i))
    full = lambda a: pl.BlockSpec(a.shape, lambda i: (0,) * a.ndim)
    wspecs = []
    for w, b in zip(ws, bs):
        wspecs += [full(w), full(b)]

    out = pl.pallas_call(
        _tc_body,
        grid=(grid,),
        in_specs=[col_spec, col_spec, col_spec, col_spec]
                 + wspecs + [full(wpg), full(wpm), full(bp2)],
        out_specs=pl.BlockSpec((1, _BLK), lambda i: (0, i)),
        out_shape=jax.ShapeDtypeStruct((1, BATCH), jnp.float32),
    )(g_ug, g_ig, g_um, g_im,
      ws[0], bs[0], ws[1], bs[1], ws[2], bs[2], ws[3], bs[3],
      wpg, wpm, bp2)
    return out.reshape(BATCH, 1)
